# Initial kernel scaffold; baseline (speedup 1.0000x reference)
import jax
import jax.numpy as jnp
from jax.experimental import pallas as pl


def kernel(x, w):
    raise NotImplementedError("write your pallas kernel here")


# speedup vs baseline: 13.8491x; 13.8491x over previous
import jax
import jax.numpy as jnp
from jax import lax
from jax.experimental import pallas as pl
from jax.experimental.pallas import tpu as pltpu
from jax.experimental.pallas import tpu_sc as plsc

BATCH = 4096
UNITS = 256
DIM = 64

NC = 2
NS = 16
L = 16
NW = NC * NS
ROWS = BATCH // NW
CHUNKS = UNITS // L
HALF = CHUNKS // 2


def _tile_body(x_hbm, wt_hbm, out_hbm, x_v, wt_v, out_v):
    wid = lax.axis_index("s") * NC + lax.axis_index("c")
    base = wid * ROWS
    pltpu.sync_copy(x_hbm.at[pl.ds(base, ROWS)], x_v)
    pltpu.sync_copy(wt_hbm, wt_v)

    def row_step(i, carry):
        for half in range(CHUNKS // HALF):
            def d_step(dc, accs):
                xd = x_v[i, pl.ds(dc * L, L)]
                for dl in range(L):
                    d = dc * L + dl
                    xs = xd[dl]
                    new = []
                    for c in range(HALF):
                        cc = half * HALF + c
                        v = wt_v[d, pl.ds(cc * L, L)] + xs
                        amax, amin = accs[c]
                        new.append((jnp.maximum(amax, v),
                                    jnp.minimum(amin, v)))
                    accs = tuple(new)
                return accs

            init = tuple(
                (jnp.full((L,), -jnp.inf, jnp.float32),
                 jnp.full((L,), jnp.inf, jnp.float32))
                for _ in range(HALF))
            accs = lax.fori_loop(0, DIM // L, d_step, init)
            for c in range(HALF):
                cc = half * HALF + c
                amax, amin = accs[c]
                out_v[i, pl.ds(cc * L, L)] = amax - amin
        return carry

    lax.fori_loop(0, ROWS, row_step, 0)
    pltpu.sync_copy(out_v, out_hbm.at[pl.ds(base, ROWS)])


def kernel(x, w):
    wt = w.T
    mesh = plsc.VectorSubcoreMesh(
        core_axis_name="c", subcore_axis_name="s",
        num_cores=NC, num_subcores=NS)
    f = pl.kernel(
        _tile_body,
        out_type=jax.ShapeDtypeStruct((BATCH, UNITS), jnp.float32),
        mesh=mesh,
        scratch_types=[
            pltpu.VMEM((ROWS, DIM), jnp.float32),
            pltpu.VMEM((DIM, UNITS), jnp.float32),
            pltpu.VMEM((ROWS, UNITS), jnp.float32),
        ],
    )
    return f(x, wt)


# speedup vs baseline: 37.9760x; 2.7421x over previous
import jax
import jax.numpy as jnp
from jax import lax
from jax.experimental import pallas as pl
from jax.experimental.pallas import tpu as pltpu
from jax.experimental.pallas import tpu_sc as plsc

BATCH = 4096
UNITS = 256
DIM = 64

NC = 2
NS = 16
L = 16
NW = NC * NS
ROWS = BATCH // NW
CHUNKS = UNITS // L
HALF = CHUNKS // 2


def _tile_body(x_hbm, wt_hbm, out_hbm, x_v, wt_v, out_v):
    wid = lax.axis_index("s") * NC + lax.axis_index("c")
    base = wid * ROWS
    pltpu.sync_copy(x_hbm.at[pl.ds(base * DIM, ROWS * DIM)], x_v)
    pltpu.sync_copy(wt_hbm, wt_v)

    def row_step(i, carry):
        for half in range(CHUNKS // HALF):
            def d_step(d, accs):
                xd = x_v[pl.ds(i * DIM + (d // L) * L, L)]
                xb = xd.at[jnp.full((L,), d % L, jnp.int32)].get(
                    mode="promise_in_bounds")
                new = []
                for c in range(HALF):
                    cc = half * HALF + c
                    v = wt_v[d, pl.ds(cc * L, L)] + xb
                    amax, amin = accs[c]
                    new.append((jnp.maximum(amax, v),
                                jnp.minimum(amin, v)))
                return tuple(new)

            init = tuple(
                (jnp.full((L,), -jnp.inf, jnp.float32),
                 jnp.full((L,), jnp.inf, jnp.float32))
                for _ in range(HALF))
            accs = lax.fori_loop(0, DIM, d_step, init)
            for c in range(HALF):
                cc = half * HALF + c
                amax, amin = accs[c]
                out_v[i, pl.ds(cc * L, L)] = amax - amin
        return carry

    lax.fori_loop(0, ROWS, row_step, 0)
    pltpu.sync_copy(out_v, out_hbm.at[pl.ds(base, ROWS)])


def kernel(x, w):
    wt = w.T
    mesh = plsc.VectorSubcoreMesh(
        core_axis_name="c", subcore_axis_name="s",
        num_cores=NC, num_subcores=NS)
    f = pl.kernel(
        _tile_body,
        out_type=jax.ShapeDtypeStruct((BATCH, UNITS), jnp.float32),
        mesh=mesh,
        scratch_types=[
            pltpu.VMEM((ROWS * DIM,), jnp.float32),
            pltpu.VMEM((DIM, UNITS), jnp.float32),
            pltpu.VMEM((ROWS, UNITS), jnp.float32),
        ],
    )
    return f(x.reshape(-1), wt)


# speedup vs baseline: 39.9277x; 1.0514x over previous
import jax
import jax.numpy as jnp
from jax import lax
from jax.experimental import pallas as pl
from jax.experimental.pallas import tpu as pltpu
from jax.experimental.pallas import tpu_sc as plsc

BATCH = 4096
UNITS = 256
DIM = 64

NC = 2
NS = 16
L = 16
NW = NC * NS
ROWS = BATCH // NW
CHUNKS = UNITS // L
HALF = CHUNKS


def _tile_body(x_hbm, wt_hbm, out_hbm, x_v, wt_v, out_v):
    wid = lax.axis_index("s") * NC + lax.axis_index("c")
    base = wid * ROWS
    pltpu.sync_copy(x_hbm.at[pl.ds(base * DIM, ROWS * DIM)], x_v)
    pltpu.sync_copy(wt_hbm, wt_v)

    def row_step(i, carry):
        for half in range(CHUNKS // HALF):
            def d_step(d, accs):
                xd = x_v[pl.ds(i * DIM + (d // L) * L, L)]
                xb = xd.at[jnp.full((L,), d % L, jnp.int32)].get(
                    mode="promise_in_bounds")
                new = []
                for c in range(HALF):
                    cc = half * HALF + c
                    v = wt_v[d, pl.ds(cc * L, L)] + xb
                    amax, amin = accs[c]
                    new.append((jnp.maximum(amax, v),
                                jnp.minimum(amin, v)))
                return tuple(new)

            init = tuple(
                (jnp.full((L,), -jnp.inf, jnp.float32),
                 jnp.full((L,), jnp.inf, jnp.float32))
                for _ in range(HALF))
            accs = lax.fori_loop(0, DIM, d_step, init)
            for c in range(HALF):
                cc = half * HALF + c
                amax, amin = accs[c]
                out_v[i, pl.ds(cc * L, L)] = amax - amin
        return carry

    lax.fori_loop(0, ROWS, row_step, 0)
    pltpu.sync_copy(out_v, out_hbm.at[pl.ds(base, ROWS)])


def kernel(x, w):
    wt = w.T
    mesh = plsc.VectorSubcoreMesh(
        core_axis_name="c", subcore_axis_name="s",
        num_cores=NC, num_subcores=NS)
    f = pl.kernel(
        _tile_body,
        out_type=jax.ShapeDtypeStruct((BATCH, UNITS), jnp.float32),
        mesh=mesh,
        scratch_types=[
            pltpu.VMEM((ROWS * DIM,), jnp.float32),
            pltpu.VMEM((DIM, UNITS), jnp.float32),
            pltpu.VMEM((ROWS, UNITS), jnp.float32),
        ],
    )
    return f(x.reshape(-1), wt)


# speedup vs baseline: 66.4628x; 1.6646x over previous
import jax
import jax.numpy as jnp
from jax import lax
from jax.experimental import pallas as pl
from jax.experimental.pallas import tpu as pltpu
from jax.experimental.pallas import tpu_sc as plsc

BATCH = 4096
UNITS = 256
DIM = 64

B_SC = 1024
B_TC = BATCH - B_SC

NC = 2
NS = 16
L = 16
NW = NC * NS
ROWS = B_SC // NW
CHUNKS = UNITS // L

TB = 64


def _sc_tile_body(x_hbm, wt_hbm, out_hbm, x_v, wt_v, out_v):
    wid = lax.axis_index("s") * NC + lax.axis_index("c")
    base = wid * ROWS
    pltpu.sync_copy(x_hbm.at[pl.ds(base * DIM, ROWS * DIM)], x_v)
    pltpu.sync_copy(wt_hbm, wt_v)

    def row_step(i, carry):
        def d_step(d, accs):
            xd = x_v[pl.ds(i * DIM + (d // L) * L, L)]
            xb = xd.at[jnp.full((L,), d % L, jnp.int32)].get(
                mode="promise_in_bounds")
            new = []
            for c in range(CHUNKS):
                v = wt_v[d, pl.ds(c * L, L)] + xb
                amax, amin = accs[c]
                new.append((jnp.maximum(amax, v), jnp.minimum(amin, v)))
            return tuple(new)

        init = tuple(
            (jnp.full((L,), -jnp.inf, jnp.float32),
             jnp.full((L,), jnp.inf, jnp.float32))
            for _ in range(CHUNKS))
        accs = lax.fori_loop(0, DIM, d_step, init)
        for c in range(CHUNKS):
            amax, amin = accs[c]
            out_v[i, pl.ds(c * L, L)] = amax - amin
        return carry

    lax.fori_loop(0, ROWS, row_step, 0)
    pltpu.sync_copy(out_v, out_hbm.at[pl.ds(base, ROWS)])


def _sc_part(x_sc, wt):
    mesh = plsc.VectorSubcoreMesh(
        core_axis_name="c", subcore_axis_name="s",
        num_cores=NC, num_subcores=NS)
    f = pl.kernel(
        _sc_tile_body,
        out_type=jax.ShapeDtypeStruct((B_SC, UNITS), jnp.float32),
        mesh=mesh,
        scratch_types=[
            pltpu.VMEM((ROWS * DIM,), jnp.float32),
            pltpu.VMEM((DIM, UNITS), jnp.float32),
            pltpu.VMEM((ROWS, UNITS), jnp.float32),
        ],
    )
    return f(x_sc.reshape(-1), wt)


def _tc_block_body(x_ref, wt_ref, o_ref):
    neg = jnp.full((TB, UNITS), -jnp.inf, jnp.float32)
    pos = jnp.full((TB, UNITS), jnp.inf, jnp.float32)
    amax, amin = neg, pos
    for d in range(DIM):
        v = x_ref[:, d][:, None] + wt_ref[d, :][None, :]
        amax = jnp.maximum(amax, v)
        amin = jnp.minimum(amin, v)
    o_ref[...] = amax - amin


def _tc_part(x_tc, wt):
    return pl.pallas_call(
        _tc_block_body,
        grid=(B_TC // TB,),
        in_specs=[
            pl.BlockSpec((TB, DIM), lambda i: (i, 0)),
            pl.BlockSpec((DIM, UNITS), lambda i: (0, 0)),
        ],
        out_specs=pl.BlockSpec((TB, UNITS), lambda i: (i, 0)),
        out_shape=jax.ShapeDtypeStruct((B_TC, UNITS), jnp.float32),
    )(x_tc, wt)


def kernel(x, w):
    wt = w.T
    out_sc = _sc_part(x[:B_SC], wt)
    out_tc = _tc_part(x[B_SC:], wt)
    return jnp.concatenate([out_sc, out_tc], axis=0)


# speedup vs baseline: 70.2491x; 1.0570x over previous
import jax
import jax.numpy as jnp
from jax import lax
from jax.experimental import pallas as pl
from jax.experimental.pallas import tpu as pltpu
from jax.experimental.pallas import tpu_sc as plsc

BATCH = 4096
UNITS = 256
DIM = 64

B_SC = 1280
B_TC = BATCH - B_SC

NC = 2
NS = 16
L = 16
NW = NC * NS
ROWS = B_SC // NW
CHUNKS = UNITS // L

TB = 64


def _sc_tile_body(x_hbm, wt_hbm, out_hbm, x_v, wt_v, out_v):
    wid = lax.axis_index("s") * NC + lax.axis_index("c")
    base = wid * ROWS
    pltpu.sync_copy(x_hbm.at[pl.ds(base, ROWS)], x_v)
    pltpu.sync_copy(wt_hbm, wt_v)

    def row_step(i, carry):
        def d_step(d, accs):
            xd = x_v[i, pl.ds((d // L) * L, L)]
            xb = xd.at[jnp.full((L,), d % L, jnp.int32)].get(
                mode="promise_in_bounds")
            new = []
            for c in range(CHUNKS):
                v = wt_v[d, pl.ds(c * L, L)] + xb
                amax, amin = accs[c]
                new.append((jnp.maximum(amax, v), jnp.minimum(amin, v)))
            return tuple(new)

        init = tuple(
            (jnp.full((L,), -jnp.inf, jnp.float32),
             jnp.full((L,), jnp.inf, jnp.float32))
            for _ in range(CHUNKS))
        accs = lax.fori_loop(0, DIM, d_step, init)
        for c in range(CHUNKS):
            amax, amin = accs[c]
            out_v[i, pl.ds(c * L, L)] = amax - amin
        return carry

    lax.fori_loop(0, ROWS, row_step, 0)
    pltpu.sync_copy(out_v, out_hbm.at[pl.ds(base, ROWS)])


def _sc_part(x, wt):
    mesh = plsc.VectorSubcoreMesh(
        core_axis_name="c", subcore_axis_name="s",
        num_cores=NC, num_subcores=NS)
    f = pl.kernel(
        _sc_tile_body,
        out_type=jax.ShapeDtypeStruct((B_SC, UNITS), jnp.float32),
        mesh=mesh,
        scratch_types=[
            pltpu.VMEM((ROWS, DIM), jnp.float32),
            pltpu.VMEM((DIM, UNITS), jnp.float32),
            pltpu.VMEM((ROWS, UNITS), jnp.float32),
        ],
    )
    return f(x, wt)


def _tc_block_body(x_ref, wt_ref, o_ref):
    amax = jnp.full((TB, UNITS), -jnp.inf, jnp.float32)
    amin = jnp.full((TB, UNITS), jnp.inf, jnp.float32)
    for d in range(DIM):
        v = x_ref[:, d][:, None] + wt_ref[d, :][None, :]
        amax = jnp.maximum(amax, v)
        amin = jnp.minimum(amin, v)
    o_ref[...] = amax - amin


def _tc_part(x, wt):
    off = B_SC // TB
    return pl.pallas_call(
        _tc_block_body,
        grid=(B_TC // TB,),
        in_specs=[
            pl.BlockSpec((TB, DIM), lambda i: (i + off, 0)),
            pl.BlockSpec((DIM, UNITS), lambda i: (0, 0)),
        ],
        out_specs=pl.BlockSpec((TB, UNITS), lambda i: (i, 0)),
        out_shape=jax.ShapeDtypeStruct((B_TC, UNITS), jnp.float32),
    )(x, wt)


def kernel(x, w):
    wt = w.T
    out_sc = _sc_part(x, wt)
    out_tc = _tc_part(x, wt)
    return jnp.concatenate([out_sc, out_tc], axis=0)
